# baseline (speedup 1.0000x reference)
import jax
import jax.numpy as jnp
from jax import lax
from jax.experimental import pallas as pl
from jax.experimental.pallas import tpu as pltpu
from jax.experimental.pallas import tpu_sc as plsc

_D = 32
_NC = 2
_NS = 16
_NW = _NC * _NS
_C = 128
_NB = 8
_NF = 26
_BT_PER_W = 4
_NBLK = _NF * _BT_PER_W


def _transpose_block(rows, tbuf, b):
    iota = lax.iota(jnp.int32, 16)
    for d in range(_D):
        ti, din = d // 8, d % 8
        col = jnp.full((16,), d, jnp.int32)
        for k in range(_C // 16):
            ridx = iota + (16 * k)
            v = plsc.load_gather(rows.at[b], [ridx, col])
            tbuf[b, ti, din, pl.ds(16 * k, 16)] = v


def _emb_body(table_hbm, xt_hbm, out_hbm, idx_v, rows_v, tbuf_v, gsem, osem):
    wid = lax.axis_index("s") * _NC + lax.axis_index("c")
    pltpu.sync_copy(xt_hbm.at[:, pl.ds(wid * (_BT_PER_W * _C), _BT_PER_W * _C)], idx_v)

    def idx_slice(j):
        f = j // _BT_PER_W
        bt = j % _BT_PER_W
        return idx_v.at[f, pl.ds(bt * _C, _C)], f, bt

    def start_gather(j, b):
        sl, _, _ = idx_slice(j)
        pltpu.async_copy(table_hbm.at[sl], rows_v.at[b], gsem.at[b])

    def wait_gather(j, b):
        sl, _, _ = idx_slice(j)
        pltpu.make_async_copy(table_hbm.at[sl], rows_v.at[b], gsem.at[b]).wait()

    def out_ref(j):
        f = j // _BT_PER_W
        bt = j % _BT_PER_W
        return out_hbm.at[f, :, wid * _BT_PER_W + bt]

    for b in range(_NB):
        start_gather(b, b)

    def body(j, carry):
        b = jnp.bitwise_and(j, _NB - 1)
        wait_gather(j, b)

        @pl.when(j >= _NB)
        def _():
            pltpu.make_async_copy(out_ref(j), tbuf_v.at[b], osem.at[b]).wait()

        _transpose_block(rows_v, tbuf_v, b)
        pltpu.async_copy(tbuf_v.at[b], out_ref(j), osem.at[b])

        @pl.when(j + _NB < _NBLK)
        def _():
            start_gather(j + _NB, b)

        return carry

    lax.fori_loop(0, _NBLK, body, 0)

    def drain(b, carry):
        pltpu.make_async_copy(out_ref(0), tbuf_v.at[b], osem.at[b]).wait()
        return carry

    lax.fori_loop(0, _NB, drain, 0)


def kernel(x, embedding_weight):
    b0, b1 = x.shape
    xt = jnp.transpose(x.astype(jnp.int32))
    mesh = plsc.VectorSubcoreMesh(core_axis_name="c", subcore_axis_name="s")
    run = pl.kernel(
        _emb_body,
        mesh=mesh,
        out_type=jax.ShapeDtypeStruct((_NF, _D // 8, b0 // _C, 8, _C), jnp.float32),
        scratch_types=[
            pltpu.VMEM((_NF, _BT_PER_W * _C), jnp.int32),
            pltpu.VMEM((_NB, _C, _D), jnp.float32),
            pltpu.VMEM((_NB, _D // 8, 8, _C), jnp.float32),
            pltpu.SemaphoreType.DMA((_NB,)),
            pltpu.SemaphoreType.DMA((_NB,)),
        ],
        compiler_params=pltpu.CompilerParams(
            use_tc_tiling_on_sc=False, needs_layout_passes=False
        ),
    )
    out5d = run(embedding_weight, xt)
    return out5d.transpose(2, 4, 0, 1, 3).reshape(b0, b1, _D)


# speedup vs baseline: 1.2059x; 1.2059x over previous
import jax
import jax.numpy as jnp
from jax import lax
from jax.experimental import pallas as pl
from jax.experimental.pallas import tpu as pltpu
from jax.experimental.pallas import tpu_sc as plsc

_D = 32
_NC = 2
_NS = 16
_NW = _NC * _NS
_C = 128
_NB = 4
_NF = 26
_BT_PER_W = 4
_NBLK = _NF * _BT_PER_W
_WCHUNK = _BT_PER_W * _C


def _transpose_block(rows, idx, tbuf, b, f, btbase):
    iota = lax.iota(jnp.int32, 16)
    ridx = [iota + 16 * k for k in range(_C // 16)]
    offv = [
        jnp.bitwise_and(idx[f, pl.ds(btbase + 16 * k, 16)], 3) * _D
        for k in range(_C // 16)
    ]

    @plsc.parallel_loop(0, _D, unroll=4)
    def _(d):
        ti = jnp.right_shift(d, 3)
        din = jnp.bitwise_and(d, 7)
        for k in range(_C // 16):
            v = plsc.load_gather(rows.at[b], [ridx[k], offv[k] + d])
            tbuf[b, ti, din, pl.ds(16 * k, 16)] = v


def _emb_body(table_hbm, xt_hbm, out_hbm, idx_v, q_v, rows_v, tbuf_v, gsem, osem):
    wid = lax.axis_index("s") * _NC + lax.axis_index("c")
    pltpu.sync_copy(xt_hbm.at[:, pl.ds(wid * _WCHUNK, _WCHUNK)], idx_v)

    @plsc.parallel_loop(0, _NF * _WCHUNK // 16, unroll=8)
    def _(i):
        f = i // (_WCHUNK // 16)
        col = (i % (_WCHUNK // 16)) * 16
        q_v[f, pl.ds(col, 16)] = jnp.right_shift(idx_v[f, pl.ds(col, 16)], 2)

    def start_gather(j, b):
        f = j // _BT_PER_W
        bt = j % _BT_PER_W
        pltpu.async_copy(
            table_hbm.at[q_v.at[f, pl.ds(bt * _C, _C)]], rows_v.at[b], gsem.at[b]
        )

    def wait_gather(j, b):
        f = j // _BT_PER_W
        bt = j % _BT_PER_W
        pltpu.make_async_copy(
            table_hbm.at[q_v.at[f, pl.ds(bt * _C, _C)]], rows_v.at[b], gsem.at[b]
        ).wait()

    def out_ref(j):
        f = j // _BT_PER_W
        bt = j % _BT_PER_W
        return out_hbm.at[f, :, wid * _BT_PER_W + bt]

    for b in range(_NB):
        start_gather(b, b)

    def body(j, carry):
        b = jnp.bitwise_and(j, _NB - 1)
        f = j // _BT_PER_W
        bt = j % _BT_PER_W
        wait_gather(j, b)

        @pl.when(j >= _NB)
        def _():
            pltpu.make_async_copy(out_ref(j), tbuf_v.at[b], osem.at[b]).wait()

        _transpose_block(rows_v, idx_v, tbuf_v, b, f, bt * _C)
        pltpu.async_copy(tbuf_v.at[b], out_ref(j), osem.at[b])

        @pl.when(j + _NB < _NBLK)
        def _():
            start_gather(j + _NB, b)

        return carry

    lax.fori_loop(0, _NBLK, body, 0)

    def drain(b, carry):
        pltpu.make_async_copy(out_ref(0), tbuf_v.at[b], osem.at[b]).wait()
        return carry

    lax.fori_loop(0, _NB, drain, 0)


def kernel(x, embedding_weight):
    b0, b1 = x.shape
    xt = jnp.transpose(x.astype(jnp.int32))
    t128 = embedding_weight.reshape(250000, 128)
    mesh = plsc.VectorSubcoreMesh(core_axis_name="c", subcore_axis_name="s")
    run = pl.kernel(
        _emb_body,
        mesh=mesh,
        out_type=jax.ShapeDtypeStruct((_NF, _D // 8, b0 // _C, 8, _C), jnp.float32),
        scratch_types=[
            pltpu.VMEM((_NF, _WCHUNK), jnp.int32),
            pltpu.VMEM((_NF, _WCHUNK), jnp.int32),
            pltpu.VMEM((_NB, _C, 4 * _D), jnp.float32),
            pltpu.VMEM((_NB, _D // 8, 8, _C), jnp.float32),
            pltpu.SemaphoreType.DMA((_NB,)),
            pltpu.SemaphoreType.DMA((_NB,)),
        ],
        compiler_params=pltpu.CompilerParams(
            use_tc_tiling_on_sc=False, needs_layout_passes=False
        ),
    )
    out5d = run(t128, xt)
    return out5d.transpose(2, 4, 0, 1, 3).reshape(b0, b1, _D)


# speedup vs baseline: 1.3659x; 1.1327x over previous
import jax
import jax.numpy as jnp
from jax import lax
from jax.experimental import pallas as pl
from jax.experimental.pallas import tpu as pltpu
from jax.experimental.pallas import tpu_sc as plsc

_V = 1000000
_D = 32
_NC = 2
_NS = 16
_NW = _NC * _NS
_C = 128
_NB = 4
_NF = 26
_BT_PER_W = 4
_NBLK = _NF * _BT_PER_W
_WCHUNK = _BT_PER_W * _C

_CW = 512
_NFULL = _V // _CW
_REM = _V - _NFULL * _CW


def _transpose_chunk(in_v, out_v, b, width):
    iota = lax.iota(jnp.int32, 16)
    dvecs = [iota + d0 for d0 in (0, 16)]
    zeros = jnp.zeros((16,), jnp.int32)

    @plsc.parallel_loop(0, width, unroll=8)
    def _(r):
        rv = zeros + r
        g = jnp.right_shift(r, 2)
        col = jnp.bitwise_and(r, 3) * _D
        for half, d0 in enumerate((0, 16)):
            v = plsc.load_gather(in_v.at[b], [dvecs[half], rv])
            out_v[b, g, pl.ds(col + d0, 16)] = v


def _relayout_body(tt_hbm, rem_hbm, out_hbm, in_v, out_v, isem, osem):
    wid = lax.axis_index("s") * _NC + lax.axis_index("c")

    def start_in(t, b):
        ci = wid + _NW * t
        pltpu.async_copy(
            tt_hbm.at[:, pl.ds(ci * _CW, _CW)], in_v.at[b], isem.at[b]
        )

    def wait_in(b):
        pltpu.make_async_copy(
            tt_hbm.at[:, pl.ds(0, _CW)], in_v.at[b], isem.at[b]
        ).wait()

    def start_out(t, b):
        ci = wid + _NW * t
        pltpu.async_copy(
            out_v.at[b], out_hbm.at[pl.ds(ci * (_CW // 4), _CW // 4)], osem.at[b]
        )

    def wait_out(b):
        pltpu.make_async_copy(
            out_hbm.at[pl.ds(0, _CW // 4)], out_v.at[b], osem.at[b]
        ).wait()

    nt = (_NFULL - wid + _NW - 1) // _NW

    @pl.when(nt > 0)
    def _():
        start_in(0, 0)

    @pl.when(nt > 1)
    def _():
        start_in(1, 1)

    def body(t, carry):
        b = jnp.bitwise_and(t, 1)
        wait_in(b)

        @pl.when(t >= 2)
        def _():
            wait_out(b)

        _transpose_chunk(in_v, out_v, b, _CW)
        start_out(t, b)

        @pl.when(t + 2 < nt)
        def _():
            start_in(t + 2, b)

        return carry

    lax.fori_loop(0, nt, body, 0)

    @pl.when(nt > 0)
    def _():
        wait_out(0)

    @pl.when(nt > 1)
    def _():
        wait_out(1)

    @pl.when(wid == 0)
    def _():
        pltpu.sync_copy(rem_hbm, out_v.at[0, pl.ds(0, _REM // 4)])
        pltpu.sync_copy(
            out_v.at[0, pl.ds(0, _REM // 4)],
            out_hbm.at[pl.ds((_V - _REM) // 4, _REM // 4)],
        )


def _transpose_block(rows, idx, tbuf, b, f, btbase):
    iota = lax.iota(jnp.int32, 16)
    ridx = [iota + 16 * k for k in range(_C // 16)]
    offv = [
        jnp.bitwise_and(idx[f, pl.ds(btbase + 16 * k, 16)], 3) * _D
        for k in range(_C // 16)
    ]

    @plsc.parallel_loop(0, _D, unroll=4)
    def _(d):
        ti = jnp.right_shift(d, 3)
        din = jnp.bitwise_and(d, 7)
        for k in range(_C // 16):
            v = plsc.load_gather(rows.at[b], [ridx[k], offv[k] + d])
            tbuf[b, ti, din, pl.ds(16 * k, 16)] = v


def _emb_body(table_hbm, xt_hbm, out_hbm, idx_v, q_v, rows_v, tbuf_v, gsem, osem):
    wid = lax.axis_index("s") * _NC + lax.axis_index("c")
    pltpu.sync_copy(xt_hbm.at[:, pl.ds(wid * _WCHUNK, _WCHUNK)], idx_v)

    @plsc.parallel_loop(0, _NF * _WCHUNK // 16, unroll=8)
    def _(i):
        f = i // (_WCHUNK // 16)
        col = (i % (_WCHUNK // 16)) * 16
        q_v[f, pl.ds(col, 16)] = jnp.right_shift(idx_v[f, pl.ds(col, 16)], 2)

    def start_gather(j, b):
        f = j // _BT_PER_W
        bt = j % _BT_PER_W
        pltpu.async_copy(
            table_hbm.at[q_v.at[f, pl.ds(bt * _C, _C)]], rows_v.at[b], gsem.at[b]
        )

    def wait_gather(j, b):
        f = j // _BT_PER_W
        bt = j % _BT_PER_W
        pltpu.make_async_copy(
            table_hbm.at[q_v.at[f, pl.ds(bt * _C, _C)]], rows_v.at[b], gsem.at[b]
        ).wait()

    def out_ref(j):
        f = j // _BT_PER_W
        bt = j % _BT_PER_W
        return out_hbm.at[f, :, wid * _BT_PER_W + bt]

    for b in range(_NB):
        start_gather(b, b)

    def body(j, carry):
        b = jnp.bitwise_and(j, _NB - 1)
        f = j // _BT_PER_W
        bt = j % _BT_PER_W
        wait_gather(j, b)

        @pl.when(j >= _NB)
        def _():
            pltpu.make_async_copy(out_ref(j), tbuf_v.at[b], osem.at[b]).wait()

        _transpose_block(rows_v, idx_v, tbuf_v, b, f, bt * _C)
        pltpu.async_copy(tbuf_v.at[b], out_ref(j), osem.at[b])

        @pl.when(j + _NB < _NBLK)
        def _():
            start_gather(j + _NB, b)

        return carry

    lax.fori_loop(0, _NBLK, body, 0)

    def drain(b, carry):
        pltpu.make_async_copy(out_ref(0), tbuf_v.at[b], osem.at[b]).wait()
        return carry

    lax.fori_loop(0, _NB, drain, 0)


def kernel(x, embedding_weight):
    b0, b1 = x.shape
    xt = jnp.transpose(x.astype(jnp.int32))
    tt = jnp.transpose(embedding_weight)
    rem128 = embedding_weight[_V - _REM:].reshape(_REM // 4, 4 * _D)
    mesh = plsc.VectorSubcoreMesh(core_axis_name="c", subcore_axis_name="s")
    params = pltpu.CompilerParams(use_tc_tiling_on_sc=True, needs_layout_passes=False)

    relayout = pl.kernel(
        _relayout_body,
        mesh=mesh,
        out_type=jax.ShapeDtypeStruct((_V // 4, 4 * _D), jnp.float32),
        scratch_types=[
            pltpu.VMEM((2, _D, _CW), jnp.float32),
            pltpu.VMEM((2, _CW // 4, 4 * _D), jnp.float32),
            pltpu.SemaphoreType.DMA((2,)),
            pltpu.SemaphoreType.DMA((2,)),
        ],
        compiler_params=params,
    )
    t128 = relayout(tt, rem128)

    gather = pl.kernel(
        _emb_body,
        mesh=mesh,
        out_type=jax.ShapeDtypeStruct((_NF, _D // 8, b0 // _C, 8, _C), jnp.float32),
        scratch_types=[
            pltpu.VMEM((_NF, _WCHUNK), jnp.int32),
            pltpu.VMEM((_NF, _WCHUNK), jnp.int32),
            pltpu.VMEM((_NB, _C, 4 * _D), jnp.float32),
            pltpu.VMEM((_NB, _D // 8, 8, _C), jnp.float32),
            pltpu.SemaphoreType.DMA((_NB,)),
            pltpu.SemaphoreType.DMA((_NB,)),
        ],
        compiler_params=params,
    )
    out5d = gather(t128, xt)
    return out5d.transpose(2, 4, 0, 1, 3).reshape(b0, b1, _D)


# speedup vs baseline: 1.3733x; 1.0055x over previous
import jax
import jax.numpy as jnp
from jax import lax
from jax.experimental import pallas as pl
from jax.experimental.pallas import tpu as pltpu
from jax.experimental.pallas import tpu_sc as plsc

_V = 1000000
_D = 32
_NC = 2
_NS = 16
_NW = _NC * _NS
_C = 128
_NB = 4
_NF = 26
_BT_PER_W = 4
_NBLK = _NF * _BT_PER_W
_WCHUNK = _BT_PER_W * _C

_CW = 512
_NFULL = _V // _CW
_REM = _V - _NFULL * _CW


def _transpose_chunk(in_v, out_v, b, width):
    iota = lax.iota(jnp.int32, 16)
    dvecs = [iota + d0 for d0 in (0, 16)]
    zeros = jnp.zeros((16,), jnp.int32)

    @plsc.parallel_loop(0, width, unroll=8)
    def _(r):
        rv = zeros + r
        g = jnp.right_shift(r, 2)
        col = jnp.bitwise_and(r, 3) * _D
        for half, d0 in enumerate((0, 16)):
            v = plsc.load_gather(in_v.at[b], [dvecs[half], rv])
            out_v[b, g, pl.ds(col + d0, 16)] = v


def _relayout_body(tt_hbm, rem_hbm, out_hbm, in_v, out_v, isem, osem):
    wid = lax.axis_index("s") * _NC + lax.axis_index("c")

    def start_in(t, b):
        ci = wid + _NW * t
        pltpu.async_copy(
            tt_hbm.at[:, pl.ds(ci * _CW, _CW)], in_v.at[b], isem.at[b]
        )

    def wait_in(b):
        pltpu.make_async_copy(
            tt_hbm.at[:, pl.ds(0, _CW)], in_v.at[b], isem.at[b]
        ).wait()

    def start_out(t, b):
        ci = wid + _NW * t
        pltpu.async_copy(
            out_v.at[b], out_hbm.at[pl.ds(ci * (_CW // 4), _CW // 4)], osem.at[b]
        )

    def wait_out(b):
        pltpu.make_async_copy(
            out_hbm.at[pl.ds(0, _CW // 4)], out_v.at[b], osem.at[b]
        ).wait()

    nt = (_NFULL - wid + _NW - 1) // _NW

    @pl.when(nt > 0)
    def _():
        start_in(0, 0)

    @pl.when(nt > 1)
    def _():
        start_in(1, 1)

    def body(t, carry):
        b = jnp.bitwise_and(t, 1)
        wait_in(b)

        @pl.when(t >= 2)
        def _():
            wait_out(b)

        _transpose_chunk(in_v, out_v, b, _CW)
        start_out(t, b)

        @pl.when(t + 2 < nt)
        def _():
            start_in(t + 2, b)

        return carry

    lax.fori_loop(0, nt, body, 0)

    @pl.when(nt > 0)
    def _():
        wait_out(0)

    @pl.when(nt > 1)
    def _():
        wait_out(1)

    @pl.when(wid == 0)
    def _():
        pltpu.sync_copy(rem_hbm, out_v.at[0, pl.ds(0, _REM // 4)])
        pltpu.sync_copy(
            out_v.at[0, pl.ds(0, _REM // 4)],
            out_hbm.at[pl.ds((_V - _REM) // 4, _REM // 4)],
        )


def _transpose_block(rows, tbuf, b):
    iota = lax.iota(jnp.int32, 16)
    ridx = [iota + 16 * k for k in range(_C // 16)]
    zeros = jnp.zeros((16,), jnp.int32)

    @plsc.parallel_loop(0, _D, unroll=4)
    def _(d):
        ti = jnp.right_shift(d, 3)
        din = jnp.bitwise_and(d, 7)
        dv = zeros + d
        for k in range(_C // 16):
            v = plsc.load_gather(rows.at[b], [ridx[k], dv])
            tbuf[b, ti, din, pl.ds(16 * k, 16)] = v


def _emb_body(table_hbm, xt_hbm, out_hbm, idx_v, rows_v, tbuf_v, gsem, osem):
    wid = lax.axis_index("s") * _NC + lax.axis_index("c")
    pltpu.sync_copy(xt_hbm.at[:, pl.ds(wid * _WCHUNK, _WCHUNK)], idx_v)

    def start_gather(j, b):
        f = j // _BT_PER_W
        bt = j % _BT_PER_W
        pltpu.async_copy(
            table_hbm.at[idx_v.at[f, pl.ds(bt * _C, _C)]], rows_v.at[b], gsem.at[b]
        )

    def wait_gather(j, b):
        f = j // _BT_PER_W
        bt = j % _BT_PER_W
        pltpu.make_async_copy(
            table_hbm.at[idx_v.at[f, pl.ds(bt * _C, _C)]], rows_v.at[b], gsem.at[b]
        ).wait()

    def out_ref(j):
        f = j // _BT_PER_W
        bt = j % _BT_PER_W
        return out_hbm.at[f, :, wid * _BT_PER_W + bt]

    for b in range(_NB):
        start_gather(b, b)

    def body(j, carry):
        b = jnp.bitwise_and(j, _NB - 1)
        f = j // _BT_PER_W
        bt = j % _BT_PER_W
        wait_gather(j, b)

        @pl.when(j >= _NB)
        def _():
            pltpu.make_async_copy(out_ref(j), tbuf_v.at[b], osem.at[b]).wait()

        _transpose_block(rows_v, tbuf_v, b)
        pltpu.async_copy(tbuf_v.at[b], out_ref(j), osem.at[b])

        @pl.when(j + _NB < _NBLK)
        def _():
            start_gather(j + _NB, b)

        return carry

    lax.fori_loop(0, _NBLK, body, 0)

    def drain(b, carry):
        pltpu.make_async_copy(out_ref(0), tbuf_v.at[b], osem.at[b]).wait()
        return carry

    lax.fori_loop(0, _NB, drain, 0)


def kernel(x, embedding_weight):
    b0, b1 = x.shape
    xt = jnp.transpose(x.astype(jnp.int32))
    tt = jnp.transpose(embedding_weight)
    rem128 = embedding_weight[_V - _REM:].reshape(_REM // 4, 4 * _D)
    mesh = plsc.VectorSubcoreMesh(core_axis_name="c", subcore_axis_name="s")
    params = pltpu.CompilerParams(use_tc_tiling_on_sc=True, needs_layout_passes=False)

    relayout = pl.kernel(
        _relayout_body,
        mesh=mesh,
        out_type=jax.ShapeDtypeStruct((_V // 4, 4 * _D), jnp.float32),
        scratch_types=[
            pltpu.VMEM((2, _D, _CW), jnp.float32),
            pltpu.VMEM((2, _CW // 4, 4 * _D), jnp.float32),
            pltpu.SemaphoreType.DMA((2,)),
            pltpu.SemaphoreType.DMA((2,)),
        ],
        compiler_params=params,
    )
    t1m = relayout(tt, rem128).reshape(_V, _D)

    gather = pl.kernel(
        _emb_body,
        mesh=mesh,
        out_type=jax.ShapeDtypeStruct((_NF, _D // 8, b0 // _C, 8, _C), jnp.float32),
        scratch_types=[
            pltpu.VMEM((_NF, _WCHUNK), jnp.int32),
            pltpu.VMEM((_NB, _C, _D), jnp.float32),
            pltpu.VMEM((_NB, _D // 8, 8, _C), jnp.float32),
            pltpu.SemaphoreType.DMA((_NB,)),
            pltpu.SemaphoreType.DMA((_NB,)),
        ],
        compiler_params=pltpu.CompilerParams(
            use_tc_tiling_on_sc=False, needs_layout_passes=False
        ),
    )
    out5d = gather(t1m, xt)
    return out5d.transpose(2, 4, 0, 1, 3).reshape(b0, b1, _D)
